# Initial kernel scaffold; baseline (speedup 1.0000x reference)
import jax
import jax.numpy as jnp
from jax.experimental import pallas as pl


def kernel(features, edge_index, edge_weight, W1, b1, ln1_g, ln1_b, W2, b2, ln2_g, ln2_b, W3, b3, ln3_g, ln3_b, fc_W, fc_b):
    raise NotImplementedError("write your pallas kernel here")


# speedup vs baseline: 1.5671x; 1.5671x over previous
import functools

import jax
import jax.numpy as jnp
from jax import lax
from jax.experimental import pallas as pl
from jax.experimental.pallas import tpu as pltpu
from jax.experimental.pallas import tpu_sc as plsc

N = 50000
HID = 64
NT = 16
NC = 2
EB = 1024
GCH = 64
NROW = EB // GCH
NBATCH = 50
EPAD = NT * EB * NBATCH
R = EPAD // GCH
RPT = R // NT
NPADN = 50176
ROWS_PER_TILE = NPADN // NT
NPAD_DEG = NT * 3200

@functools.lru_cache(maxsize=None)
def _mesh():
    return plsc.VectorSubcoreMesh(core_axis_name="c", subcore_axis_name="s",
                                  num_cores=NC, num_subcores=NT)


def _deg_body(src_hbm, dst_hbm, vals_hbm, out_hbm, idx_v, vals_v, zbuf, acc):
    c = lax.axis_index("c")
    s = lax.axis_index("s")
    tb = s * 3200

    def zero(i, _):
        zbuf[pl.ds(i * 16, 16)] = jnp.zeros((16,), jnp.float32)
        return 0

    lax.fori_loop(0, 200, zero, 0)
    pltpu.sync_copy(zbuf, acc.at[pl.ds(tb, 3200)])
    plsc.subcore_barrier()

    def batch(b, _):
        r0 = s * RPT + b * NROW

        @pl.when(c == 0)
        def _():
            pltpu.sync_copy(src_hbm.at[pl.ds(r0, NROW)], idx_v)

        @pl.when(c == 1)
        def _():
            pltpu.sync_copy(dst_hbm.at[pl.ds(r0, NROW)], idx_v)

        pltpu.sync_copy(vals_hbm.at[pl.ds(r0, NROW)], vals_v)
        for j in range(NROW):
            pltpu.sync_copy(vals_v.at[j], acc.at[idx_v.at[j]], add=True)
        return 0

    lax.fori_loop(0, NBATCH, batch, 0)
    plsc.subcore_barrier()
    pltpu.sync_copy(acc.at[pl.ds(tb, 3200)], out_hbm.at[c, pl.ds(tb, 3200)])


@functools.lru_cache(maxsize=None)
def _deg_call():
    return pl.kernel(
        _deg_body,
        out_type=jax.ShapeDtypeStruct((NC, NPAD_DEG), jnp.float32),
        mesh=_mesh(),
        scratch_types=[
            pltpu.VMEM((NROW, GCH), jnp.int32),
            pltpu.VMEM((NROW, GCH), jnp.float32),
            pltpu.VMEM((3200,), jnp.float32),
            pltpu.VMEM_SHARED((NPAD_DEG,), jnp.float32),
        ],
    )


def _make_spmm_body(col0):
    def _spmm_body(tab_hbm, src_hbm, dst_hbm, ew_hbm, nidx_hbm, out_hbm,
                   src_v, dst_v, ew_v, rows_v, msg_v, nidx_v, acc, sem, sem2):
        c = lax.axis_index("c")
        s = lax.axis_index("s")
        tb = s * ROWS_PER_TILE
        eb0 = s * (RPT * GCH)
        cb = col0 + c * 32

        for i in range(GCH):
            msg_v[i, pl.ds(0, 16)] = jnp.zeros((16,), jnp.float32)
            msg_v[i, pl.ds(16, 16)] = jnp.zeros((16,), jnp.float32)
        pltpu.sync_copy(nidx_hbm.at[pl.ds(s * 56, 56)], nidx_v)
        for q in range(49):
            pltpu.sync_copy(msg_v, acc.at[nidx_v.at[q]])
        plsc.subcore_barrier()

        def batch(b, _):
            r0 = s * RPT + b * NROW
            pltpu.sync_copy(src_hbm.at[pl.ds(r0, NROW)], src_v)
            pltpu.sync_copy(dst_hbm.at[pl.ds(r0, NROW)], dst_v)
            pltpu.sync_copy(ew_hbm.at[pl.ds(eb0 + b * EB, EB)], ew_v)

            for j in range(NROW):
                pltpu.async_copy(tab_hbm.at[src_v.at[j]], rows_v, sem).wait()

                def mul(g, _):
                    ew16 = ew_v[pl.ds(j * GCH + g * 16, 16)]
                    for i in range(16):
                        k = g * 16 + i
                        w = ew16[i]
                        a = rows_v[k, pl.ds(cb, 16)]
                        msg_v[k, pl.ds(0, 16)] = a * w
                        b2 = rows_v[k, pl.ds(cb + 16, 16)]
                        msg_v[k, pl.ds(16, 16)] = b2 * w
                    return 0

                lax.fori_loop(0, GCH // 16, mul, 0)
                pltpu.sync_copy(msg_v, acc.at[dst_v.at[j]], add=True)
            return 0

        lax.fori_loop(0, NBATCH, batch, 0)
        plsc.subcore_barrier()
        for q in range(49):
            pltpu.async_copy(acc.at[nidx_v.at[q]], msg_v, sem2).wait()
            pltpu.sync_copy(msg_v, out_hbm.at[c, pl.ds(tb + q * GCH, GCH)])

    return _spmm_body


@functools.lru_cache(maxsize=None)
def _spmm_call(col0):
    return pl.kernel(
        _make_spmm_body(col0),
        out_type=jax.ShapeDtypeStruct((NC, NPADN, 32), jnp.float32),
        mesh=_mesh(),
        scratch_types=[
            pltpu.VMEM((NROW, GCH), jnp.int32),
            pltpu.VMEM((NROW, GCH), jnp.int32),
            pltpu.VMEM((EB,), jnp.float32),
            pltpu.VMEM((GCH, 128), jnp.float32),
            pltpu.VMEM((GCH, 32), jnp.float32),
            pltpu.VMEM((56, GCH), jnp.int32),
            pltpu.VMEM_SHARED((NPADN, 32), jnp.float32),
            pltpu.SemaphoreType.DMA,
            pltpu.SemaphoreType.DMA,
        ],
    )


BN = 400
GRID = N // BN
EPS = 1e-5


def _prep_body(x_ref, od_ref, id_ref, y_ref, ns_ref, nd_ref):
    ns = lax.rsqrt(jnp.maximum(od_ref[...], 1.0))
    nd = lax.rsqrt(jnp.maximum(id_ref[...], 1.0))
    h = x_ref[...] * ns
    y_ref[...] = jnp.concatenate([h, jnp.zeros_like(h)], axis=1)
    ns_ref[...] = ns
    nd_ref[...] = nd


def _layer_norm(y, g, b):
    mu = jnp.mean(y, axis=-1, keepdims=True)
    var = jnp.mean((y - mu) ** 2, axis=-1, keepdims=True)
    return (y - mu) * lax.rsqrt(var + EPS) * g + b


def _tc1_body(a0, a1, nd, ns, W, b, g, be, o):
    agg = jnp.concatenate([a0[...], a1[...]], axis=1) * nd[...]
    y = jnp.dot(agg, W[...], preferred_element_type=jnp.float32) + b[...]
    o[...] = jnp.maximum(_layer_norm(y, g[...], be[...]), 0.0) * ns[...]


def _tc2_body(a0, a1, a2, a3, nd, ns, W, b, g, be, W3, o):
    agg = jnp.concatenate([a0[...], a1[...], a2[...], a3[...]], axis=1) * nd[...]
    y = jnp.dot(agg, W[...], preferred_element_type=jnp.float32) + b[...]
    h = jnp.maximum(_layer_norm(y, g[...], be[...]), 0.0) * ns[...]
    z = jnp.dot(h, W3[...], preferred_element_type=jnp.float32)
    o[...] = jnp.concatenate([z, jnp.zeros_like(z)], axis=1)


def _tc3_body(a0, a1, nd, b, g, be, fcw, fcb, out):
    i = pl.program_id(0)
    z = jnp.concatenate([a0[...], a1[...]], axis=1) * nd[...] + b[...]
    h = jnp.maximum(_layer_norm(z, g[...], be[...]), 0.0)
    part = jnp.sum(h, axis=0, keepdims=True) / N
    p = jnp.dot(part, fcw[...], preferred_element_type=jnp.float32)

    @pl.when(i == 0)
    def _():
        out[...] = p + fcb[...]

    @pl.when(i != 0)
    def _():
        out[...] = out[...] + p


def _row_spec(w):
    return pl.BlockSpec((BN, w), lambda i: (i, 0))


def _full_spec(shape):
    return pl.BlockSpec(shape, lambda i: tuple(0 for _ in shape))


_prep = pl.pallas_call(
    _prep_body,
    grid=(GRID,),
    in_specs=[_row_spec(HID), _row_spec(1), _row_spec(1)],
    out_specs=[_row_spec(2 * HID), _row_spec(1), _row_spec(1)],
    out_shape=[
        jax.ShapeDtypeStruct((N, 2 * HID), jnp.float32),
        jax.ShapeDtypeStruct((N, 1), jnp.float32),
        jax.ShapeDtypeStruct((N, 1), jnp.float32),
    ],
)

_tc1 = pl.pallas_call(
    _tc1_body,
    grid=(GRID,),
    in_specs=[_row_spec(32), _row_spec(32), _row_spec(1), _row_spec(1),
              _full_spec((HID, 2 * HID)), _full_spec((1, 2 * HID)),
              _full_spec((1, 2 * HID)), _full_spec((1, 2 * HID))],
    out_specs=_row_spec(2 * HID),
    out_shape=jax.ShapeDtypeStruct((N, 2 * HID), jnp.float32),
)

_tc2 = pl.pallas_call(
    _tc2_body,
    grid=(GRID,),
    in_specs=[_row_spec(32)] * 4 + [_row_spec(1), _row_spec(1),
              _full_spec((2 * HID, 2 * HID)), _full_spec((1, 2 * HID)),
              _full_spec((1, 2 * HID)), _full_spec((1, 2 * HID)),
              _full_spec((2 * HID, HID))],
    out_specs=_row_spec(2 * HID),
    out_shape=jax.ShapeDtypeStruct((N, 2 * HID), jnp.float32),
)

_tc3 = pl.pallas_call(
    _tc3_body,
    grid=(GRID,),
    in_specs=[_row_spec(32), _row_spec(32), _row_spec(1),
              _full_spec((1, HID)), _full_spec((1, HID)),
              _full_spec((1, HID)), _full_spec((HID, 2)),
              _full_spec((1, 2))],
    out_specs=pl.BlockSpec((1, 2), lambda i: (0, 0)),
    out_shape=jax.ShapeDtypeStruct((1, 2), jnp.float32),
)


def kernel(features, edge_index, edge_weight, W1, b1, ln1_g, ln1_b,
           W2, b2, ln2_g, ln2_b, W3, b3, ln3_g, ln3_b, fc_W, fc_b):
    E = edge_weight.shape[0]
    pad = EPAD - E
    src = edge_index[0]
    dst = edge_index[1]
    pad_idx = jnp.arange(pad, dtype=jnp.int32) % N
    src2d = jnp.concatenate([src, pad_idx]).reshape(R, GCH)
    dst2d = jnp.concatenate([dst, pad_idx]).reshape(R, GCH)
    zero_pad = jnp.zeros((pad,), jnp.float32)
    ew_flat = jnp.concatenate([edge_weight, zero_pad])
    ones2d = jnp.concatenate(
        [jnp.ones((E,), jnp.float32), zero_pad]).reshape(R, GCH)

    tiles = jnp.arange(NT)[:, None, None]
    qs = jnp.arange(56)[None, :, None]
    lanes = jnp.arange(GCH)[None, None, :]
    nidx = tiles * ROWS_PER_TILE + qs * GCH + lanes
    nidx = jnp.where(qs < 49, nidx, 0).reshape(NT * 56, GCH).astype(jnp.int32)

    ones_e = jnp.ones((E,), jnp.float32)
    outdeg = jnp.zeros((N,), jnp.float32).at[src].add(ones_e).reshape(N, 1)
    indeg = jnp.zeros((N,), jnp.float32).at[dst].add(ones_e).reshape(N, 1)

    y, ns, nd = _prep(features, outdeg, indeg)

    agg1 = _spmm_call(0)(y, src2d, dst2d, ew_flat, nidx)
    h = _tc1(agg1[0], agg1[1], nd, ns,
             W1, b1.reshape(1, -1),
             ln1_g.reshape(1, -1), ln1_b.reshape(1, -1))

    aggA = _spmm_call(0)(h, src2d, dst2d, ew_flat, nidx)
    aggB = _spmm_call(64)(h, src2d, dst2d, ew_flat, nidx)
    z = _tc2(aggA[0], aggA[1], aggB[0], aggB[1], nd, ns,
             W2, b2.reshape(1, -1),
             ln2_g.reshape(1, -1), ln2_b.reshape(1, -1), W3)

    aggC = _spmm_call(0)(z, src2d, dst2d, ew_flat, nidx)
    out = _tc3(aggC[0], aggC[1], nd,
               b3.reshape(1, -1), ln3_g.reshape(1, -1), ln3_b.reshape(1, -1),
               fc_W, fc_b.reshape(1, -1))
    return out
